# baseline (device time: 122079 ns/iter reference)
import jax
import jax.numpy as jnp
from jax import lax
from jax.experimental import pallas as pl
from jax.experimental.pallas import tpu as pltpu

N_DEV = 4
SQ = 1024
HL = 8
DH = 128
WIN = 128
KV_WIN = SQ + WIN
QB = 256
KB = 512
CA = 640
A1 = 384
D_LOC = HL * DH
D_MODEL = 1024
SCALE = 0.08838834764831843


def kernel(x, Wq, K_ext, V_ext, Wo):
    def body(x_ref, wq_ref, k_ref, v_ref, wo_ref, out_ref,
             kv_bf, stag, kgvg, q_bf, ctx_bf, ar_send, ar_recv, relay_buf,
             kv_send_sems, kv_recv_sems, ar_send_sems, ar_recv_sems,
             copy_sems, relay_sems, fwd_send_sems):
        my = lax.axis_index("i")

        bsem = pltpu.get_barrier_semaphore()
        for j in range(N_DEV):
            @pl.when(my != j)
            def _(j=j):
                pl.semaphore_signal(bsem, inc=1, device_id=(j,),
                                    device_id_type=pl.DeviceIdType.MESH)
        pl.semaphore_wait(bsem, N_DEV - 1)

        def rc(src_ref, dst_ref, send_sem, recv_sem, dev):
            return pltpu.make_async_remote_copy(
                src_ref=src_ref, dst_ref=dst_ref, send_sem=send_sem,
                recv_sem=recv_sem, device_id=(dev,),
                device_id_type=pl.DeviceIdType.MESH)

        kv_sends0 = [
            rc(kv_bf.at[2, 0, 0:CA], relay_buf.at[0:CA],
               kv_send_sems.at[0], relay_sems.at[0], 1),
            rc(kv_bf.at[2, 1, 0:CA], relay_buf.at[0:CA],
               kv_send_sems.at[1], relay_sems.at[0], 3),
            rc(kv_bf.at[2, 0, CA:SQ], relay_buf.at[CA:SQ],
               kv_send_sems.at[2], relay_sems.at[1], 1),
            rc(kv_bf.at[2, 1, CA:SQ], relay_buf.at[CA:SQ],
               kv_send_sems.at[3], relay_sems.at[1], 3),
            rc(kv_bf.at[0, :, 0:A1], kgvg.at[:, 0:A1],
               kv_send_sems.at[4], kv_recv_sems.at[0], 1),
            rc(kv_bf.at[1, :, 0:A1], kgvg.at[:, 0:A1],
               kv_send_sems.at[5], kv_recv_sems.at[0], 3),
            rc(kv_bf.at[0, :, A1:CA], kgvg.at[:, A1:CA],
               kv_send_sems.at[6], kv_recv_sems.at[4], 1),
            rc(kv_bf.at[1, :, A1:CA], kgvg.at[:, A1:CA],
               kv_send_sems.at[7], kv_recv_sems.at[4], 3),
            rc(kv_bf.at[0, :, CA:SQ], kgvg.at[:, CA:SQ],
               kv_send_sems.at[8], kv_recv_sems.at[7], 1),
            rc(kv_bf.at[1, :, CA:SQ], kgvg.at[:, CA:SQ],
               kv_send_sems.at[9], kv_recv_sems.at[7], 3),
        ]
        kv_sends1 = [rc(kv_bf.at[jj, :, 0:WIN], kgvg.at[:, SQ:KV_WIN],
                        kv_send_sems.at[jj], kv_recv_sems.at[1], j)
                     for jj, j in enumerate((0, 2, 3))]
        fwds = {
            1: [rc(relay_buf.at[0:CA], kgvg.at[0, 0:CA],
                   fwd_send_sems.at[0], kv_recv_sems.at[2], 2),
                rc(relay_buf.at[CA:SQ], kgvg.at[0, CA:SQ],
                   fwd_send_sems.at[1], kv_recv_sems.at[5], 2)],
            3: [rc(relay_buf.at[0:CA], kgvg.at[1, 0:CA],
                   fwd_send_sems.at[0], kv_recv_sems.at[3], 2),
                rc(relay_buf.at[CA:SQ], kgvg.at[1, CA:SQ],
                   fwd_send_sems.at[1], kv_recv_sems.at[6], 2)],
        }

        units0 = [
            ('kv', 2, 0, 0, CA, 2 * HL, 0),
            ('kv', 2, 1, 0, CA, 2 * HL, 1),
            ('kv', 2, 0, CA, SQ, 2 * HL, 2),
            ('kv', 2, 1, CA, SQ, 2 * HL, 3),
            ('kv', 0, 0, 0, A1, 1 * HL, None),
            ('kv', 0, 1, 0, A1, 1 * HL, 4),
            ('kv', 1, 0, 0, A1, 3 * HL, None),
            ('kv', 1, 1, 0, A1, 3 * HL, 5),
            ('kv', 0, 0, A1, CA, 1 * HL, None),
            ('kv', 0, 1, A1, CA, 1 * HL, 6),
            ('kv', 1, 0, A1, CA, 3 * HL, None),
            ('kv', 1, 1, A1, CA, 3 * HL, 7),
            ('kv', 0, 0, CA, SQ, 1 * HL, None),
            ('kv', 0, 1, CA, SQ, 1 * HL, 8),
            ('kv', 1, 0, CA, SQ, 3 * HL, None),
            ('kv', 1, 1, CA, SQ, 3 * HL, 9),
            ('kg', None, 0, 0, CA, 0, None),
            ('kg', None, 0, CA, SQ, 0, None),
            ('kg', None, 1, 0, CA, 0, None),
            ('kg', None, 1, CA, SQ, 0, None),
        ]

        def start_unit_load(u, s):
            _, _, t, r0, r1, h0, _ = u
            src = (k_ref, v_ref)[t]
            cp = pltpu.make_async_copy(
                src.at[0, r0:r1, h0:h0 + HL, :],
                stag.at[s, 0:r1 - r0], copy_sems.at[s])
            cp.start()
            return cp

        @pl.when(my == 0)
        def _():
            cur = start_unit_load(units0[0], 0)
            for i, u in enumerate(units0):
                nxt = (start_unit_load(units0[i + 1], (i + 1) % 2)
                       if i + 1 < len(units0) else None)
                cur.wait()
                kind, slot, t, r0, r1, _, send_idx = u
                data = stag[i % 2, 0:r1 - r0].astype(jnp.bfloat16)
                if kind == 'kv':
                    kv_bf[slot, t, r0:r1] = data
                else:
                    kgvg[t, r0:r1] = data
                if send_idx is not None:
                    kv_sends0[send_idx].start()
                cur = nxt

        @pl.when(my == 1)
        def _():
            for jj, j in enumerate((0, 2, 3)):
                for t, src in enumerate((k_ref, v_ref)):
                    cp = pltpu.make_async_copy(
                        src.at[0, 0:WIN, j * HL:(j + 1) * HL, :],
                        stag.at[0, 0:WIN], copy_sems.at[0])
                    cp.start()
                    cp.wait()
                    kv_bf[jj, t, 0:WIN] = stag[0, 0:WIN].astype(jnp.bfloat16)
                kv_sends1[jj].start()
            for t, src in enumerate((k_ref, v_ref)):
                cp = pltpu.make_async_copy(
                    src.at[0, 0:WIN, HL:2 * HL, :],
                    stag.at[0, 0:WIN], copy_sems.at[0])
                cp.start()
                cp.wait()
                kgvg[t, SQ:KV_WIN] = stag[0, 0:WIN].astype(jnp.bfloat16)

        x_bf = x_ref[0].astype(jnp.bfloat16)
        wq_bf = wq_ref[...].astype(jnp.bfloat16)
        q = jnp.dot(x_bf, wq_bf, preferred_element_type=jnp.float32)
        q_bf[...] = q.astype(jnp.bfloat16)

        def wait_bytes(dst_ref, recv_sem):
            pltpu.make_async_remote_copy(
                src_ref=dst_ref, dst_ref=dst_ref,
                send_sem=fwd_send_sems.at[0], recv_sem=recv_sem,
                device_id=(0,), device_id_type=pl.DeviceIdType.MESH,
            ).wait_recv()

        for relayer in (1, 3):
            @pl.when(my == relayer)
            def _(relayer=relayer):
                wait_bytes(relay_buf.at[0:CA], relay_sems.at[0])
                fwds[relayer][0].start()
                wait_bytes(relay_buf.at[CA:SQ], relay_sems.at[1])
                fwds[relayer][1].start()

        wo_bf = wo_ref[...].astype(jnp.bfloat16)
        p1 = my ^ 1
        p2 = my ^ 3
        NB = SQ // QB

        def make_ex(step, b, partner):
            return pltpu.make_async_remote_copy(
                src_ref=ar_send.at[step, b], dst_ref=ar_recv.at[step, b],
                send_sem=ar_send_sems.at[step * NB + b],
                recv_sem=ar_recv_sems.at[step * NB + b],
                device_id=(partner,), device_id_type=pl.DeviceIdType.MESH)

        partials = [None] * NB
        sums = [None] * NB
        ex0 = [None] * NB
        ex1 = [None] * NB

        def start_step1(b):
            ex0[b].wait()
            sums[b] = partials[b] + ar_recv[0, b].astype(jnp.float32)
            ar_send[1, b] = sums[b].astype(jnp.bfloat16)
            ex1[b] = make_ex(1, b, p2)
            ex1[b].start()

        def finish_block(b):
            ex1[b].wait()
            out_ref[0, b * QB:(b + 1) * QB, :] = (
                sums[b] + ar_recv[1, b].astype(jnp.float32))

        for b in range(NB):
            if b in (0, 1, 2):
                dsem, rows = {
                    0: (0, slice(0, A1)),
                    1: (4, slice(A1, CA)),
                    2: (7, slice(CA, SQ)),
                }[b]

                @pl.when((my == 1) | (my == 3))
                def _(dsem=dsem, rows=rows):
                    wait_bytes(kgvg.at[:, rows], kv_recv_sems.at[dsem])
            if b in (0, 2):
                rel_k = kv_recv_sems.at[2 if b == 0 else 5]
                rel_v = kv_recv_sems.at[3 if b == 0 else 6]
                rrows = slice(0, CA) if b == 0 else slice(CA, SQ)

                @pl.when(my == 2)
                def _(rel_k=rel_k, rel_v=rel_v, rrows=rrows):
                    wait_bytes(kgvg.at[0, rrows], rel_k)
                    wait_bytes(kgvg.at[1, rrows], rel_v)
            if b == 3:
                @pl.when(my != 1)
                def _():
                    wait_bytes(kgvg.at[:, SQ:KV_WIN], kv_recv_sems.at[1])

            lo = max(0, QB * b - WIN)
            off = QB * b - lo
            qi = lax.broadcasted_iota(jnp.int32, (QB, KB), 0) + off
            ki = lax.broadcasted_iota(jnp.int32, (QB, KB), 1)
            band = jnp.abs(qi - ki) <= WIN
            for h in range(HL):
                qh = q_bf[b * QB:(b + 1) * QB, h * DH:(h + 1) * DH]
                kh = kgvg[0, lo:lo + KB, h, :]
                s = lax.dot_general(qh, kh, (((1,), (1,)), ((), ())),
                                    preferred_element_type=jnp.float32) * SCALE
                s = jnp.where(band, s, -1e9)
                m = jnp.max(s, axis=1, keepdims=True)
                e = jnp.exp(s - m)
                w = (e / jnp.sum(e, axis=1, keepdims=True)).astype(jnp.bfloat16)
                vh = kgvg[1, lo:lo + KB, h, :]
                ctx_h = jnp.dot(w, vh, preferred_element_type=jnp.float32)
                ctx_bf[b * QB:(b + 1) * QB, h * DH:(h + 1) * DH] = (
                    ctx_h.astype(jnp.bfloat16))
            partials[b] = jnp.dot(ctx_bf[b * QB:(b + 1) * QB, :], wo_bf,
                                  preferred_element_type=jnp.float32)
            ar_send[0, b] = partials[b].astype(jnp.bfloat16)
            ex0[b] = make_ex(0, b, p1)
            ex0[b].start()
            if b >= 1:
                start_step1(b - 1)
            if b >= 2:
                finish_block(b - 2)

        start_step1(NB - 1)
        finish_block(NB - 2)
        finish_block(NB - 1)

        @pl.when(my == 0)
        def _():
            for r in kv_sends0:
                r.wait_send()

        @pl.when(my == 1)
        def _():
            for r in kv_sends1:
                r.wait_send()

        for relayer in (1, 3):
            @pl.when(my == relayer)
            def _(relayer=relayer):
                for r in fwds[relayer]:
                    r.wait_send()

    return pl.pallas_call(
        body,
        out_shape=jax.ShapeDtypeStruct((1, SQ, D_MODEL), jnp.float32),
        in_specs=[
            pl.BlockSpec(memory_space=pltpu.VMEM),
            pl.BlockSpec(memory_space=pltpu.VMEM),
            pl.BlockSpec(memory_space=pl.ANY),
            pl.BlockSpec(memory_space=pl.ANY),
            pl.BlockSpec(memory_space=pltpu.VMEM),
        ],
        out_specs=pl.BlockSpec(memory_space=pltpu.VMEM),
        scratch_shapes=[
            pltpu.VMEM((3, 2, SQ, HL, DH), jnp.bfloat16),
            pltpu.VMEM((2, CA, HL, DH), jnp.float32),
            pltpu.VMEM((2, KV_WIN, HL, DH), jnp.bfloat16),
            pltpu.VMEM((SQ, D_LOC), jnp.bfloat16),
            pltpu.VMEM((SQ, D_LOC), jnp.bfloat16),
            pltpu.VMEM((2, 4, QB, D_MODEL), jnp.bfloat16),
            pltpu.VMEM((2, 4, QB, D_MODEL), jnp.bfloat16),
            pltpu.VMEM((SQ, HL, DH), jnp.bfloat16),
            pltpu.SemaphoreType.DMA((10,)),
            pltpu.SemaphoreType.DMA((8,)),
            pltpu.SemaphoreType.DMA((8,)),
            pltpu.SemaphoreType.DMA((8,)),
            pltpu.SemaphoreType.DMA((2,)),
            pltpu.SemaphoreType.DMA((2,)),
            pltpu.SemaphoreType.DMA((2,)),
        ],
        compiler_params=pltpu.CompilerParams(
            collective_id=0, vmem_limit_bytes=60 * 1024 * 1024),
    )(x, Wq, K_ext, V_ext, Wo)


# device time: 122071 ns/iter; 1.0001x vs baseline; 1.0001x over previous
import jax
import jax.numpy as jnp
from jax import lax
from jax.experimental import pallas as pl
from jax.experimental.pallas import tpu as pltpu

N_DEV = 4
SQ = 1024
HL = 8
DH = 128
WIN = 128
KV_WIN = SQ + WIN
QB = 256
KB = 512
CA = 640
A1 = 384
D_LOC = HL * DH
D_MODEL = 1024
SCALE = 0.08838834764831843


def kernel(x, Wq, K_ext, V_ext, Wo):
    def body(x_ref, wq_ref, k_ref, v_ref, wo_ref, out_ref,
             kv_bf, stag, kgvg, q_bf, ctx_bf, ar_send, ar_recv, relay_buf,
             kv_send_sems, kv_recv_sems, ar_send_sems, ar_recv_sems,
             copy_sems, relay_sems, fwd_send_sems):
        my = lax.axis_index("i")

        bsem = pltpu.get_barrier_semaphore()
        for j in range(N_DEV):
            @pl.when(my != j)
            def _(j=j):
                pl.semaphore_signal(bsem, inc=1, device_id=(j,),
                                    device_id_type=pl.DeviceIdType.MESH)
        pl.semaphore_wait(bsem, N_DEV - 1)

        def rc(src_ref, dst_ref, send_sem, recv_sem, dev):
            return pltpu.make_async_remote_copy(
                src_ref=src_ref, dst_ref=dst_ref, send_sem=send_sem,
                recv_sem=recv_sem, device_id=(dev,),
                device_id_type=pl.DeviceIdType.MESH)

        kv_sends0 = [
            rc(kv_bf.at[2, 0, 0:CA], relay_buf.at[0:CA],
               kv_send_sems.at[0], relay_sems.at[0], 1),
            rc(kv_bf.at[2, 1, 0:CA], relay_buf.at[0:CA],
               kv_send_sems.at[1], relay_sems.at[0], 3),
            rc(kv_bf.at[2, 0, CA:SQ], relay_buf.at[CA:SQ],
               kv_send_sems.at[2], relay_sems.at[1], 1),
            rc(kv_bf.at[2, 1, CA:SQ], relay_buf.at[CA:SQ],
               kv_send_sems.at[3], relay_sems.at[1], 3),
            rc(kv_bf.at[0, :, 0:A1], kgvg.at[:, 0:A1],
               kv_send_sems.at[4], kv_recv_sems.at[0], 1),
            rc(kv_bf.at[1, :, 0:A1], kgvg.at[:, 0:A1],
               kv_send_sems.at[5], kv_recv_sems.at[0], 3),
            rc(kv_bf.at[0, :, A1:CA], kgvg.at[:, A1:CA],
               kv_send_sems.at[6], kv_recv_sems.at[4], 1),
            rc(kv_bf.at[1, :, A1:CA], kgvg.at[:, A1:CA],
               kv_send_sems.at[7], kv_recv_sems.at[4], 3),
            rc(kv_bf.at[0, :, CA:SQ], kgvg.at[:, CA:SQ],
               kv_send_sems.at[8], kv_recv_sems.at[7], 1),
            rc(kv_bf.at[1, :, CA:SQ], kgvg.at[:, CA:SQ],
               kv_send_sems.at[9], kv_recv_sems.at[7], 3),
        ]
        kv_sends1 = [rc(kv_bf.at[jj, :, 0:WIN], kgvg.at[:, SQ:KV_WIN],
                        kv_send_sems.at[jj], kv_recv_sems.at[1], j)
                     for jj, j in enumerate((0, 2, 3))]
        fwds = {
            1: [rc(relay_buf.at[0:CA], kgvg.at[0, 0:CA],
                   fwd_send_sems.at[0], kv_recv_sems.at[2], 2),
                rc(relay_buf.at[CA:SQ], kgvg.at[0, CA:SQ],
                   fwd_send_sems.at[1], kv_recv_sems.at[5], 2)],
            3: [rc(relay_buf.at[0:CA], kgvg.at[1, 0:CA],
                   fwd_send_sems.at[0], kv_recv_sems.at[3], 2),
                rc(relay_buf.at[CA:SQ], kgvg.at[1, CA:SQ],
                   fwd_send_sems.at[1], kv_recv_sems.at[6], 2)],
        }

        units0 = [
            ('kv', 2, 0, 0, CA, 2 * HL, 0),
            ('kv', 2, 1, 0, CA, 2 * HL, 1),
            ('kv', 2, 0, CA, SQ, 2 * HL, 2),
            ('kv', 2, 1, CA, SQ, 2 * HL, 3),
            ('kv', 0, 0, 0, A1, 1 * HL, None),
            ('kv', 0, 1, 0, A1, 1 * HL, 4),
            ('kv', 1, 0, 0, A1, 3 * HL, None),
            ('kv', 1, 1, 0, A1, 3 * HL, 5),
            ('kv', 0, 0, A1, CA, 1 * HL, None),
            ('kv', 0, 1, A1, CA, 1 * HL, 6),
            ('kv', 1, 0, A1, CA, 3 * HL, None),
            ('kv', 1, 1, A1, CA, 3 * HL, 7),
            ('kv', 0, 0, CA, SQ, 1 * HL, None),
            ('kv', 0, 1, CA, SQ, 1 * HL, 8),
            ('kv', 1, 0, CA, SQ, 3 * HL, None),
            ('kv', 1, 1, CA, SQ, 3 * HL, 9),
            ('kg', None, 0, 0, CA, 0, None),
            ('kg', None, 0, CA, SQ, 0, None),
            ('kg', None, 1, 0, CA, 0, None),
            ('kg', None, 1, CA, SQ, 0, None),
        ]

        def start_unit_load(u, s):
            _, _, t, r0, r1, h0, _ = u
            src = (k_ref, v_ref)[t]
            cp = pltpu.make_async_copy(
                src.at[0, r0:r1, h0:h0 + HL, :],
                stag.at[s, 0:r1 - r0], copy_sems.at[s])
            cp.start()
            return cp

        @pl.when(my == 0)
        def _():
            cur = start_unit_load(units0[0], 0)
            for i, u in enumerate(units0):
                cur.wait()
                nxt = (start_unit_load(units0[i + 1], (i + 1) % 2)
                       if i + 1 < len(units0) else None)
                kind, slot, t, r0, r1, _, send_idx = u
                data = stag[i % 2, 0:r1 - r0].astype(jnp.bfloat16)
                if kind == 'kv':
                    kv_bf[slot, t, r0:r1] = data
                else:
                    kgvg[t, r0:r1] = data
                if send_idx is not None:
                    kv_sends0[send_idx].start()
                cur = nxt

        @pl.when(my == 1)
        def _():
            for jj, j in enumerate((0, 2, 3)):
                for t, src in enumerate((k_ref, v_ref)):
                    cp = pltpu.make_async_copy(
                        src.at[0, 0:WIN, j * HL:(j + 1) * HL, :],
                        stag.at[0, 0:WIN], copy_sems.at[0])
                    cp.start()
                    cp.wait()
                    kv_bf[jj, t, 0:WIN] = stag[0, 0:WIN].astype(jnp.bfloat16)
                kv_sends1[jj].start()
            for t, src in enumerate((k_ref, v_ref)):
                cp = pltpu.make_async_copy(
                    src.at[0, 0:WIN, HL:2 * HL, :],
                    stag.at[0, 0:WIN], copy_sems.at[0])
                cp.start()
                cp.wait()
                kgvg[t, SQ:KV_WIN] = stag[0, 0:WIN].astype(jnp.bfloat16)

        x_bf = x_ref[0].astype(jnp.bfloat16)
        wq_bf = wq_ref[...].astype(jnp.bfloat16)
        q = jnp.dot(x_bf, wq_bf, preferred_element_type=jnp.float32)
        q_bf[...] = q.astype(jnp.bfloat16)

        def wait_bytes(dst_ref, recv_sem):
            pltpu.make_async_remote_copy(
                src_ref=dst_ref, dst_ref=dst_ref,
                send_sem=fwd_send_sems.at[0], recv_sem=recv_sem,
                device_id=(0,), device_id_type=pl.DeviceIdType.MESH,
            ).wait_recv()

        for relayer in (1, 3):
            @pl.when(my == relayer)
            def _(relayer=relayer):
                wait_bytes(relay_buf.at[0:CA], relay_sems.at[0])
                fwds[relayer][0].start()
                wait_bytes(relay_buf.at[CA:SQ], relay_sems.at[1])
                fwds[relayer][1].start()

        wo_bf = wo_ref[...].astype(jnp.bfloat16)
        p1 = my ^ 1
        p2 = my ^ 3
        NB = SQ // QB

        def make_ex(step, b, partner):
            return pltpu.make_async_remote_copy(
                src_ref=ar_send.at[step, b], dst_ref=ar_recv.at[step, b],
                send_sem=ar_send_sems.at[step * NB + b],
                recv_sem=ar_recv_sems.at[step * NB + b],
                device_id=(partner,), device_id_type=pl.DeviceIdType.MESH)

        partials = [None] * NB
        sums = [None] * NB
        ex0 = [None] * NB
        ex1 = [None] * NB

        def start_step1(b):
            ex0[b].wait()
            sums[b] = partials[b] + ar_recv[0, b].astype(jnp.float32)
            ar_send[1, b] = sums[b].astype(jnp.bfloat16)
            ex1[b] = make_ex(1, b, p2)
            ex1[b].start()

        def finish_block(b):
            ex1[b].wait()
            out_ref[0, b * QB:(b + 1) * QB, :] = (
                sums[b] + ar_recv[1, b].astype(jnp.float32))

        for b in range(NB):
            if b in (0, 1, 2):
                dsem, rows = {
                    0: (0, slice(0, A1)),
                    1: (4, slice(A1, CA)),
                    2: (7, slice(CA, SQ)),
                }[b]

                @pl.when((my == 1) | (my == 3))
                def _(dsem=dsem, rows=rows):
                    wait_bytes(kgvg.at[:, rows], kv_recv_sems.at[dsem])
            if b in (0, 2):
                rel_k = kv_recv_sems.at[2 if b == 0 else 5]
                rel_v = kv_recv_sems.at[3 if b == 0 else 6]
                rrows = slice(0, CA) if b == 0 else slice(CA, SQ)

                @pl.when(my == 2)
                def _(rel_k=rel_k, rel_v=rel_v, rrows=rrows):
                    wait_bytes(kgvg.at[0, rrows], rel_k)
                    wait_bytes(kgvg.at[1, rrows], rel_v)
            if b == 3:
                @pl.when(my != 1)
                def _():
                    wait_bytes(kgvg.at[:, SQ:KV_WIN], kv_recv_sems.at[1])

            lo = max(0, QB * b - WIN)
            kb = QB * b + QB + WIN - lo
            off = QB * b - lo
            qi = lax.broadcasted_iota(jnp.int32, (QB, kb), 0) + off
            ki = lax.broadcasted_iota(jnp.int32, (QB, kb), 1)
            band = jnp.abs(qi - ki) <= WIN
            for h in range(HL):
                qh = q_bf[b * QB:(b + 1) * QB, h * DH:(h + 1) * DH]
                kh = kgvg[0, lo:lo + kb, h, :]
                s = lax.dot_general(qh, kh, (((1,), (1,)), ((), ())),
                                    preferred_element_type=jnp.float32) * SCALE
                s = jnp.where(band, s, -1e9)
                m = jnp.max(s, axis=1, keepdims=True)
                e = jnp.exp(s - m)
                w = (e / jnp.sum(e, axis=1, keepdims=True)).astype(jnp.bfloat16)
                vh = kgvg[1, lo:lo + kb, h, :]
                ctx_h = jnp.dot(w, vh, preferred_element_type=jnp.float32)
                ctx_bf[b * QB:(b + 1) * QB, h * DH:(h + 1) * DH] = (
                    ctx_h.astype(jnp.bfloat16))
            partials[b] = jnp.dot(ctx_bf[b * QB:(b + 1) * QB, :], wo_bf,
                                  preferred_element_type=jnp.float32)
            ar_send[0, b] = partials[b].astype(jnp.bfloat16)
            ex0[b] = make_ex(0, b, p1)
            ex0[b].start()
            if b >= 1:
                start_step1(b - 1)
            if b >= 2:
                finish_block(b - 2)

        start_step1(NB - 1)
        finish_block(NB - 2)
        finish_block(NB - 1)

        @pl.when(my == 0)
        def _():
            for r in kv_sends0:
                r.wait_send()

        @pl.when(my == 1)
        def _():
            for r in kv_sends1:
                r.wait_send()

        for relayer in (1, 3):
            @pl.when(my == relayer)
            def _(relayer=relayer):
                for r in fwds[relayer]:
                    r.wait_send()

    return pl.pallas_call(
        body,
        out_shape=jax.ShapeDtypeStruct((1, SQ, D_MODEL), jnp.float32),
        in_specs=[
            pl.BlockSpec(memory_space=pltpu.VMEM),
            pl.BlockSpec(memory_space=pltpu.VMEM),
            pl.BlockSpec(memory_space=pl.ANY),
            pl.BlockSpec(memory_space=pl.ANY),
            pl.BlockSpec(memory_space=pltpu.VMEM),
        ],
        out_specs=pl.BlockSpec(memory_space=pltpu.VMEM),
        scratch_shapes=[
            pltpu.VMEM((3, 2, SQ, HL, DH), jnp.bfloat16),
            pltpu.VMEM((2, CA, HL, DH), jnp.float32),
            pltpu.VMEM((2, KV_WIN, HL, DH), jnp.bfloat16),
            pltpu.VMEM((SQ, D_LOC), jnp.bfloat16),
            pltpu.VMEM((SQ, D_LOC), jnp.bfloat16),
            pltpu.VMEM((2, 4, QB, D_MODEL), jnp.bfloat16),
            pltpu.VMEM((2, 4, QB, D_MODEL), jnp.bfloat16),
            pltpu.VMEM((SQ, HL, DH), jnp.bfloat16),
            pltpu.SemaphoreType.DMA((10,)),
            pltpu.SemaphoreType.DMA((8,)),
            pltpu.SemaphoreType.DMA((8,)),
            pltpu.SemaphoreType.DMA((8,)),
            pltpu.SemaphoreType.DMA((2,)),
            pltpu.SemaphoreType.DMA((2,)),
            pltpu.SemaphoreType.DMA((2,)),
        ],
        compiler_params=pltpu.CompilerParams(
            collective_id=0, vmem_limit_bytes=60 * 1024 * 1024),
    )(x, Wq, K_ext, V_ext, Wo)


# device time: 119993 ns/iter; 1.0174x vs baseline; 1.0173x over previous
import jax
import jax.numpy as jnp
from jax import lax
from jax.experimental import pallas as pl
from jax.experimental.pallas import tpu as pltpu

N_DEV = 4
SQ = 1024
HL = 8
DH = 128
WIN = 128
KV_WIN = SQ + WIN
QB = 256
KB = 512
CA = 640
A1 = 384
D_LOC = HL * DH
D_MODEL = 1024
SCALE = 0.08838834764831843


def kernel(x, Wq, K_ext, V_ext, Wo):
    def body(x_ref, wq_ref, k_ref, v_ref, wo_ref, out_ref,
             kv_bf, stag, kgvg, q_bf, ctx_bf, ar_send, ar_recv, relay_buf,
             kv_send_sems, kv_recv_sems, ar_send_sems, ar_recv_sems,
             copy_sems, relay_sems, fwd_send_sems):
        my = lax.axis_index("i")

        bsem = pltpu.get_barrier_semaphore()
        for j in range(N_DEV):
            @pl.when(my != j)
            def _(j=j):
                pl.semaphore_signal(bsem, inc=1, device_id=(j,),
                                    device_id_type=pl.DeviceIdType.MESH)
        pl.semaphore_wait(bsem, N_DEV - 1)

        def rc(src_ref, dst_ref, send_sem, recv_sem, dev):
            return pltpu.make_async_remote_copy(
                src_ref=src_ref, dst_ref=dst_ref, send_sem=send_sem,
                recv_sem=recv_sem, device_id=(dev,),
                device_id_type=pl.DeviceIdType.MESH)

        kv_sends0 = [
            rc(kv_bf.at[2, 0, 0:CA], relay_buf.at[0:CA],
               kv_send_sems.at[0], relay_sems.at[0], 1),
            rc(kv_bf.at[2, 1, 0:CA], relay_buf.at[0:CA],
               kv_send_sems.at[1], relay_sems.at[0], 3),
            rc(kv_bf.at[2, 0, CA:SQ], relay_buf.at[CA:SQ],
               kv_send_sems.at[2], relay_sems.at[1], 1),
            rc(kv_bf.at[2, 1, CA:SQ], relay_buf.at[CA:SQ],
               kv_send_sems.at[3], relay_sems.at[1], 3),
            rc(kv_bf.at[0, :, 0:A1], kgvg.at[:, 0:A1],
               kv_send_sems.at[4], kv_recv_sems.at[0], 1),
            rc(kv_bf.at[1, :, 0:A1], kgvg.at[:, 0:A1],
               kv_send_sems.at[5], kv_recv_sems.at[0], 3),
            rc(kv_bf.at[0, :, A1:CA], kgvg.at[:, A1:CA],
               kv_send_sems.at[6], kv_recv_sems.at[4], 1),
            rc(kv_bf.at[1, :, A1:CA], kgvg.at[:, A1:CA],
               kv_send_sems.at[7], kv_recv_sems.at[4], 3),
            rc(kv_bf.at[0, :, CA:SQ], kgvg.at[:, CA:SQ],
               kv_send_sems.at[8], kv_recv_sems.at[7], 1),
            rc(kv_bf.at[1, :, CA:SQ], kgvg.at[:, CA:SQ],
               kv_send_sems.at[9], kv_recv_sems.at[7], 3),
        ]
        kv_sends1 = [rc(kv_bf.at[jj, :, 0:WIN], kgvg.at[:, SQ:KV_WIN],
                        kv_send_sems.at[jj], kv_recv_sems.at[1], j)
                     for jj, j in enumerate((0, 2, 3))]
        fwds = {
            1: [rc(relay_buf.at[0:CA], kgvg.at[0, 0:CA],
                   fwd_send_sems.at[0], kv_recv_sems.at[2], 2),
                rc(relay_buf.at[CA:SQ], kgvg.at[0, CA:SQ],
                   fwd_send_sems.at[1], kv_recv_sems.at[5], 2)],
            3: [rc(relay_buf.at[0:CA], kgvg.at[1, 0:CA],
                   fwd_send_sems.at[0], kv_recv_sems.at[3], 2),
                rc(relay_buf.at[CA:SQ], kgvg.at[1, CA:SQ],
                   fwd_send_sems.at[1], kv_recv_sems.at[6], 2)],
        }

        units0 = [
            ('kv', 2, 0, 0, CA, 2 * HL, 0),
            ('kv', 2, 1, 0, CA, 2 * HL, 1),
            ('kv', 2, 0, CA, SQ, 2 * HL, 2),
            ('kv', 2, 1, CA, SQ, 2 * HL, 3),
            ('kv', 0, 0, 0, A1, 1 * HL, None),
            ('kv', 0, 1, 0, A1, 1 * HL, 4),
            ('kv', 1, 0, 0, A1, 3 * HL, None),
            ('kv', 1, 1, 0, A1, 3 * HL, 5),
            ('kv', 0, 0, A1, CA, 1 * HL, None),
            ('kv', 0, 1, A1, CA, 1 * HL, 6),
            ('kv', 1, 0, A1, CA, 3 * HL, None),
            ('kv', 1, 1, A1, CA, 3 * HL, 7),
            ('kv', 0, 0, CA, SQ, 1 * HL, None),
            ('kv', 0, 1, CA, SQ, 1 * HL, 8),
            ('kv', 1, 0, CA, SQ, 3 * HL, None),
            ('kv', 1, 1, CA, SQ, 3 * HL, 9),
            ('kg', None, 0, 0, CA, 0, None),
            ('kg', None, 0, CA, SQ, 0, None),
            ('kg', None, 1, 0, CA, 0, None),
            ('kg', None, 1, CA, SQ, 0, None),
        ]

        def start_unit_load(u, s):
            _, _, t, r0, r1, h0, _ = u
            src = (k_ref, v_ref)[t]
            cp = pltpu.make_async_copy(
                src.at[0, r0:r1, h0:h0 + HL, :],
                stag.at[s, 0:r1 - r0], copy_sems.at[s])
            cp.start()
            return cp

        @pl.when(my == 0)
        def _():
            cur = start_unit_load(units0[0], 0)
            for i, u in enumerate(units0):
                cur.wait()
                nxt = (start_unit_load(units0[i + 1], (i + 1) % 2)
                       if i + 1 < len(units0) else None)
                kind, slot, t, r0, r1, _, send_idx = u
                data = stag[i % 2, 0:r1 - r0].astype(jnp.bfloat16)
                if kind == 'kv':
                    kv_bf[slot, t, r0:r1] = data
                else:
                    kgvg[t, r0:r1] = data
                if send_idx is not None:
                    kv_sends0[send_idx].start()
                cur = nxt

        @pl.when(my == 1)
        def _():
            for jj, j in enumerate((0, 2, 3)):
                for t, src in enumerate((k_ref, v_ref)):
                    cp = pltpu.make_async_copy(
                        src.at[0, 0:WIN, j * HL:(j + 1) * HL, :],
                        stag.at[0, 0:WIN], copy_sems.at[0])
                    cp.start()
                    cp.wait()
                    kv_bf[jj, t, 0:WIN] = stag[0, 0:WIN].astype(jnp.bfloat16)
                kv_sends1[jj].start()
            for t, src in enumerate((k_ref, v_ref)):
                cp = pltpu.make_async_copy(
                    src.at[0, 0:WIN, HL:2 * HL, :],
                    stag.at[0, 0:WIN], copy_sems.at[0])
                cp.start()
                cp.wait()
                kgvg[t, SQ:KV_WIN] = stag[0, 0:WIN].astype(jnp.bfloat16)

        x_bf = x_ref[0].astype(jnp.bfloat16)
        wq_bf = wq_ref[...].astype(jnp.bfloat16)
        q = jnp.dot(x_bf, wq_bf, preferred_element_type=jnp.float32)
        q_bf[...] = q.astype(jnp.bfloat16)

        def wait_bytes(dst_ref, recv_sem):
            pltpu.make_async_remote_copy(
                src_ref=dst_ref, dst_ref=dst_ref,
                send_sem=fwd_send_sems.at[0], recv_sem=recv_sem,
                device_id=(0,), device_id_type=pl.DeviceIdType.MESH,
            ).wait_recv()

        for relayer in (1, 3):
            @pl.when(my == relayer)
            def _(relayer=relayer):
                wait_bytes(relay_buf.at[0:CA], relay_sems.at[0])
                fwds[relayer][0].start()
                wait_bytes(relay_buf.at[CA:SQ], relay_sems.at[1])
                fwds[relayer][1].start()

        wo_bf = wo_ref[...].astype(jnp.bfloat16)
        p1 = my ^ 1
        p2 = my ^ 3
        NB = SQ // QB

        def make_ex(step, b, partner):
            return pltpu.make_async_remote_copy(
                src_ref=ar_send.at[step, b], dst_ref=ar_recv.at[step, b],
                send_sem=ar_send_sems.at[step * NB + b],
                recv_sem=ar_recv_sems.at[step * NB + b],
                device_id=(partner,), device_id_type=pl.DeviceIdType.MESH)

        partials = [None] * NB
        sums = [None] * NB
        ex0 = [None] * NB
        ex1 = [None] * NB

        def start_step1(b):
            ex0[b].wait()
            sums[b] = partials[b] + ar_recv[0, b].astype(jnp.float32)
            ar_send[1, b] = sums[b].astype(jnp.bfloat16)
            ex1[b] = make_ex(1, b, p2)
            ex1[b].start()

        def finish_block(b):
            ex1[b].wait()
            out_ref[0, b * QB:(b + 1) * QB, :] = (
                sums[b] + ar_recv[1, b].astype(jnp.float32))

        for b in range(NB):
            if b in (0, 1, 2):
                dsem, rows = {
                    0: (0, slice(0, A1)),
                    1: (4, slice(A1, CA)),
                    2: (7, slice(CA, SQ)),
                }[b]

                @pl.when((my == 1) | (my == 3))
                def _(dsem=dsem, rows=rows):
                    wait_bytes(kgvg.at[:, rows], kv_recv_sems.at[dsem])
            if b in (0, 2):
                rel_k = kv_recv_sems.at[2 if b == 0 else 5]
                rel_v = kv_recv_sems.at[3 if b == 0 else 6]
                rrows = slice(0, CA) if b == 0 else slice(CA, SQ)

                @pl.when(my == 2)
                def _(rel_k=rel_k, rel_v=rel_v, rrows=rrows):
                    wait_bytes(kgvg.at[0, rrows], rel_k)
                    wait_bytes(kgvg.at[1, rrows], rel_v)
            if b == 3:
                @pl.when(my != 1)
                def _():
                    wait_bytes(kgvg.at[:, SQ:KV_WIN], kv_recv_sems.at[1])

            lo = max(0, QB * b - WIN)
            kb = QB * b + QB + WIN - lo
            off = QB * b - lo
            qi = lax.broadcasted_iota(jnp.int32, (QB, kb), 0) + off
            ki = lax.broadcasted_iota(jnp.int32, (QB, kb), 1)
            bandf = (jnp.abs(qi - ki) <= WIN).astype(jnp.float32)
            for h in range(HL):
                qh = q_bf[b * QB:(b + 1) * QB, h * DH:(h + 1) * DH]
                kh = kgvg[0, lo:lo + kb, h, :]
                s = lax.dot_general(qh, kh, (((1,), (1,)), ((), ())),
                                    preferred_element_type=jnp.float32) * SCALE
                e = jnp.exp(s) * bandf
                denom = jnp.sum(e, axis=1, keepdims=True)
                vh = kgvg[1, lo:lo + kb, h, :]
                num = jnp.dot(e.astype(jnp.bfloat16), vh,
                              preferred_element_type=jnp.float32)
                ctx_h = num * (1.0 / denom)
                ctx_bf[b * QB:(b + 1) * QB, h * DH:(h + 1) * DH] = (
                    ctx_h.astype(jnp.bfloat16))
            partials[b] = jnp.dot(ctx_bf[b * QB:(b + 1) * QB, :], wo_bf,
                                  preferred_element_type=jnp.float32)
            ar_send[0, b] = partials[b].astype(jnp.bfloat16)
            ex0[b] = make_ex(0, b, p1)
            ex0[b].start()
            if b >= 1:
                start_step1(b - 1)
            if b >= 2:
                finish_block(b - 2)

        start_step1(NB - 1)
        finish_block(NB - 2)
        finish_block(NB - 1)

        @pl.when(my == 0)
        def _():
            for r in kv_sends0:
                r.wait_send()

        @pl.when(my == 1)
        def _():
            for r in kv_sends1:
                r.wait_send()

        for relayer in (1, 3):
            @pl.when(my == relayer)
            def _(relayer=relayer):
                for r in fwds[relayer]:
                    r.wait_send()

    return pl.pallas_call(
        body,
        out_shape=jax.ShapeDtypeStruct((1, SQ, D_MODEL), jnp.float32),
        in_specs=[
            pl.BlockSpec(memory_space=pltpu.VMEM),
            pl.BlockSpec(memory_space=pltpu.VMEM),
            pl.BlockSpec(memory_space=pl.ANY),
            pl.BlockSpec(memory_space=pl.ANY),
            pl.BlockSpec(memory_space=pltpu.VMEM),
        ],
        out_specs=pl.BlockSpec(memory_space=pltpu.VMEM),
        scratch_shapes=[
            pltpu.VMEM((3, 2, SQ, HL, DH), jnp.bfloat16),
            pltpu.VMEM((2, CA, HL, DH), jnp.float32),
            pltpu.VMEM((2, KV_WIN, HL, DH), jnp.bfloat16),
            pltpu.VMEM((SQ, D_LOC), jnp.bfloat16),
            pltpu.VMEM((SQ, D_LOC), jnp.bfloat16),
            pltpu.VMEM((2, 4, QB, D_MODEL), jnp.bfloat16),
            pltpu.VMEM((2, 4, QB, D_MODEL), jnp.bfloat16),
            pltpu.VMEM((SQ, HL, DH), jnp.bfloat16),
            pltpu.SemaphoreType.DMA((10,)),
            pltpu.SemaphoreType.DMA((8,)),
            pltpu.SemaphoreType.DMA((8,)),
            pltpu.SemaphoreType.DMA((8,)),
            pltpu.SemaphoreType.DMA((2,)),
            pltpu.SemaphoreType.DMA((2,)),
            pltpu.SemaphoreType.DMA((2,)),
        ],
        compiler_params=pltpu.CompilerParams(
            collective_id=0, vmem_limit_bytes=60 * 1024 * 1024),
    )(x, Wq, K_ext, V_ext, Wo)
